# baseline (speedup 1.0000x reference)
import functools

import jax
import jax.numpy as jnp
from jax import lax
from jax.experimental import pallas as pl
from jax.experimental.pallas import tpu as pltpu
from jax.experimental.pallas import tpu_sc as plsc

_B, _L, _D, _MAXS, _NSEG = 4, 4096, 768, 100, 50
_NT = 8
_TPT = _L // _NT
_TCH = 32
_NCH = _TPT // _TCH
_NTRI = (_NCH - 1) // 3
_LN = 16
_GPC = _TCH // _LN
_KV = _D // _LN
_NROW = _NSEG + 8

_mesh = plsc.VectorSubcoreMesh(core_axis_name="c", subcore_axis_name="s")


@functools.partial(
    pl.kernel,
    out_type=jax.ShapeDtypeStruct((_NT, _B, _NROW, _D // 2), jnp.int32),
    mesh=_mesh,
    scratch_types=[
        pltpu.VMEM((_TCH, _D), jnp.float32),
        pltpu.VMEM((_TCH, _D), jnp.float32),
        pltpu.VMEM((_TCH, _D), jnp.float32),
        pltpu.VMEM((_TPT,), jnp.int32),
        pltpu.VMEM((_NROW, _D), jnp.float32),
        pltpu.VMEM((8, _D // 2), jnp.int32),
        pltpu.SemaphoreType.DMA,
        pltpu.SemaphoreType.DMA,
        pltpu.SemaphoreType.DMA,
    ],
    compiler_params=pltpu.CompilerParams(needs_layout_passes=False),
)
def _sc_segmax(wf_hbm, ids_hbm, part_hbm, x0_v, x1_v, x2_v, ids_v, acc_v,
               stg_v, sem0, sem1, sem2):
    cid = lax.axis_index("c")
    sid = lax.axis_index("s")
    wid = sid * 2 + cid
    b = wid // _NT
    e = wid % _NT
    t0 = e * _TPT

    pltpu.sync_copy(ids_hbm.at[b, pl.ds(t0, _TPT)], ids_v)

    neg_inf = jnp.full((_LN,), -jnp.inf, jnp.float32)
    bufs = (x0_v, x1_v, x2_v)
    sems = (sem0, sem1, sem2)

    def _init_row(i, carry):
        for k in range(_KV):
            acc_v[i, pl.ds(k * _LN, _LN)] = neg_inf
        return carry

    lax.fori_loop(0, _NROW, _init_row, 0)

    def _src(c):
        return wf_hbm.at[b, pl.ds(t0 + c * _TCH, _TCH), :]

    def _wait(c, u):
        pltpu.make_async_copy(_src(c), bufs[u], sems[u]).wait()

    def _issue(c, u):
        pltpu.async_copy(_src(c), bufs[u], sems[u])

    def _compute(c, x_v):
        idg = [ids_v[pl.ds(c * _TCH + g * _LN, _LN)] for g in range(_GPC)]
        mn = jnp.min(idg[0])
        mx = jnp.max(idg[-1])

        def _seg(s, carry):
            sp = jnp.full((_LN,), s, jnp.int32)
            st = jnp.sum((idg[0] < sp).astype(jnp.int32))
            en = jnp.sum((idg[0] <= sp).astype(jnp.int32))
            for g in range(1, _GPC):
                st = st + jnp.sum((idg[g] < sp).astype(jnp.int32))
                en = en + jnp.sum((idg[g] <= sp).astype(jnp.int32))

            def _tok(t, accs):
                return tuple(
                    jnp.maximum(a, x_v[t, pl.ds(k * _LN, _LN)])
                    for k, a in enumerate(accs))

            accs = lax.fori_loop(st, en, _tok, (neg_inf,) * _KV)
            r = jnp.where(s == 0, _NSEG, s - 1)
            for k in range(_KV):
                col = pl.ds(k * _LN, _LN)
                acc_v[r, col] = jnp.maximum(acc_v[r, col], accs[k])
            return carry

        lax.fori_loop(mn, mx + 1, _seg, 0)

    _issue(0, 0)
    _issue(1, 1)
    _issue(2, 2)

    def _triple(ct, carry):
        c0 = 3 * ct
        for u in range(3):
            _wait(c0 + u, u)
            _compute(c0 + u, bufs[u])
            if u == 0:
                _issue(c0 + 3, 0)
            else:
                @pl.when(ct < _NTRI - 1)
                def _(u=u):
                    _issue(c0 + u + 3, u)

        return carry

    lax.fori_loop(0, _NTRI, _triple, 0)
    _wait(_NCH - 1, (_NCH - 1) % 3)
    _compute(_NCH - 1, bufs[(_NCH - 1) % 3])

    lane = lax.iota(jnp.int32, _LN)

    def _pack8(rb, carry):
        for j in range(8):
            rsp = jnp.full((_LN,), rb * 8 + j, jnp.int32)
            for k2 in range(_D // 32):
                ev = plsc.load_gather(acc_v, [rsp, k2 * 32 + 2 * lane])
                od = plsc.load_gather(acc_v, [rsp, k2 * 32 + 2 * lane + 1])
                pk = plsc.pack(ev, od, format=plsc.PackFormat.INTERLEAVED)
                stg_v[j, pl.ds(k2 * _LN, _LN)] = plsc.bitcast(pk, jnp.int32)
        pltpu.sync_copy(stg_v, part_hbm.at[e, b, pl.ds(rb * 8, 8), :])
        return carry

    lax.fori_loop(0, _NROW // 8, _pack8, 0)


def _tc_body(p_ref, we_ref, wo_ref, b_ref, ids_ref, o_ref):
    we = we_ref[...]
    wo = wo_ref[...]
    bias = b_ref[...]
    pad = jnp.broadcast_to(bias, (_MAXS - _NSEG, _D))
    j = pl.program_id(0)
    row = lax.broadcasted_iota(jnp.int32, (_NSEG, 1), 0) + 1
    for t in range(2):
        lo = lax.bitcast_convert_type(
            lax.shift_left(p_ref[0, t, :_NSEG, :], 16), jnp.float32)
        hi = lax.bitcast_convert_type(
            p_ref[0, t, :_NSEG, :] & jnp.int32(-65536), jnp.float32)
        for e in range(1, _NT):
            pe = p_ref[e, t, :_NSEG, :]
            lo = jnp.maximum(lo, lax.bitcast_convert_type(
                lax.shift_left(pe, 16), jnp.float32))
            hi = jnp.maximum(hi, lax.bitcast_convert_type(
                pe & jnp.int32(-65536), jnp.float32))
        bb = ids_ref[2 * j + t, 127]
        lo = jnp.where(row <= bb, lo, 0.0)
        hi = jnp.where(row <= bb, hi, 0.0)
        y = (lax.dot_general(lo, we, (((1,), (1,)), ((), ())),
                             preferred_element_type=jnp.float32)
             + lax.dot_general(hi, wo, (((1,), (1,)), ((), ())),
                               preferred_element_type=jnp.float32)
             + bias)
        o_ref[t * _MAXS:t * _MAXS + _NSEG, :] = y
        o_ref[t * _MAXS + _NSEG:(t + 1) * _MAXS, :] = pad


_tc_linear = pl.pallas_call(
    _tc_body,
    grid=(_B // 2,),
    in_specs=[
        pl.BlockSpec((_NT, 2, _NROW, _D // 2), lambda i: (0, i, 0, 0)),
        pl.BlockSpec((_D, _D // 2), lambda i: (0, 0)),
        pl.BlockSpec((_D, _D // 2), lambda i: (0, 0)),
        pl.BlockSpec((1, _D), lambda i: (0, 0)),
        pl.BlockSpec((_B, 128), lambda i: (0, _L // 128 - 1)),
    ],
    out_specs=pl.BlockSpec((2 * _MAXS, _D), lambda i: (i, 0)),
    out_shape=jax.ShapeDtypeStruct((_B * _MAXS, _D), jnp.float32),
)


def kernel(word_feature, sentence_mask, device, W, b):
    ids = sentence_mask.reshape(_B, _L).astype(jnp.int32)
    part = _sc_segmax(word_feature, ids)
    w_even = W[:, 0::2]
    w_odd = W[:, 1::2]
    out = _tc_linear(part, w_even, w_odd, b.reshape(1, _D), ids)
    return out.reshape(_B, _MAXS, _D)


# speedup vs baseline: 2.0369x; 2.0369x over previous
import functools

import jax
import jax.numpy as jnp
from jax import lax
from jax.experimental import pallas as pl
from jax.experimental.pallas import tpu as pltpu
from jax.experimental.pallas import tpu_sc as plsc

_B, _L, _D, _MAXS, _NSEG = 4, 4096, 768, 100, 50
_NT = 8
_TPT = _L // _NT
_TCH = 32
_NCH = _TPT // _TCH
_NTRI = (_NCH - 1) // 3
_LN = 16
_GPC = _TCH // _LN
_KV = _D // _LN
_NROW = _NSEG + 8

_mesh = plsc.VectorSubcoreMesh(core_axis_name="c", subcore_axis_name="s")


@functools.partial(
    pl.kernel,
    out_type=jax.ShapeDtypeStruct((_NT, _B, _NROW, _D), jnp.float32),
    mesh=_mesh,
    scratch_types=[
        pltpu.VMEM((_TCH, _D), jnp.float32),
        pltpu.VMEM((_TCH, _D), jnp.float32),
        pltpu.VMEM((_TCH, _D), jnp.float32),
        pltpu.VMEM((_TPT,), jnp.int32),
        pltpu.VMEM((_NROW, _D), jnp.float32),
        pltpu.SemaphoreType.DMA,
        pltpu.SemaphoreType.DMA,
        pltpu.SemaphoreType.DMA,
    ],
    compiler_params=pltpu.CompilerParams(needs_layout_passes=False),
)
def _sc_segmax(wf_hbm, ids_hbm, part_hbm, x0_v, x1_v, x2_v, ids_v, acc_v,
               sem0, sem1, sem2):
    cid = lax.axis_index("c")
    sid = lax.axis_index("s")
    wid = sid * 2 + cid
    b = wid // _NT
    e = wid % _NT
    t0 = e * _TPT

    pltpu.sync_copy(ids_hbm.at[b, pl.ds(t0, _TPT)], ids_v)

    neg_inf = jnp.full((_LN,), -jnp.inf, jnp.float32)
    bufs = (x0_v, x1_v, x2_v)
    sems = (sem0, sem1, sem2)

    def _init_row(i, carry):
        for k in range(_KV):
            acc_v[i, pl.ds(k * _LN, _LN)] = neg_inf
        return carry

    lax.fori_loop(0, _NROW, _init_row, 0)

    def _src(c):
        return wf_hbm.at[b, pl.ds(t0 + c * _TCH, _TCH), :]

    def _wait(c, u):
        pltpu.make_async_copy(_src(c), bufs[u], sems[u]).wait()

    def _issue(c, u):
        pltpu.async_copy(_src(c), bufs[u], sems[u])

    def _compute(c, x_v):
        idg = [ids_v[pl.ds(c * _TCH + g * _LN, _LN)] for g in range(_GPC)]
        mn = jnp.min(idg[0])
        mx = jnp.max(idg[-1])

        def _seg(s, carry):
            sp = jnp.full((_LN,), s, jnp.int32)
            st = jnp.sum((idg[0] < sp).astype(jnp.int32))
            en = jnp.sum((idg[0] <= sp).astype(jnp.int32))
            for g in range(1, _GPC):
                st = st + jnp.sum((idg[g] < sp).astype(jnp.int32))
                en = en + jnp.sum((idg[g] <= sp).astype(jnp.int32))

            def _tok(t, accs):
                return tuple(
                    jnp.maximum(a, x_v[t, pl.ds(k * _LN, _LN)])
                    for k, a in enumerate(accs))

            accs = lax.fori_loop(st, en, _tok, (neg_inf,) * _KV)
            r = jnp.where(s == 0, _NSEG, s - 1)
            for k in range(_KV):
                col = pl.ds(k * _LN, _LN)
                acc_v[r, col] = jnp.maximum(acc_v[r, col], accs[k])
            return carry

        lax.fori_loop(mn, mx + 1, _seg, 0)

    _issue(0, 0)
    _issue(1, 1)
    _issue(2, 2)

    def _triple(ct, carry):
        c0 = 3 * ct
        for u in range(3):
            _wait(c0 + u, u)
            _compute(c0 + u, bufs[u])
            if u == 0:
                _issue(c0 + 3, 0)
            else:
                @pl.when(ct < _NTRI - 1)
                def _(u=u):
                    _issue(c0 + u + 3, u)

        return carry

    lax.fori_loop(0, _NTRI, _triple, 0)
    _wait(_NCH - 1, (_NCH - 1) % 3)
    _compute(_NCH - 1, bufs[(_NCH - 1) % 3])

    pltpu.sync_copy(acc_v, part_hbm.at[e, b])


def _tc_body(p_ref, w_ref, b_ref, ids_ref, o_ref):
    w = w_ref[...]
    bias = b_ref[...]
    pad = jnp.broadcast_to(bias, (_MAXS - _NSEG, _D))
    j = pl.program_id(0)
    row = lax.broadcasted_iota(jnp.int32, (_NSEG, 1), 0) + 1
    for t in range(2):
        m = p_ref[0, t, :_NSEG, :]
        for e in range(1, _NT):
            m = jnp.maximum(m, p_ref[e, t, :_NSEG, :])
        bb = ids_ref[2 * j + t, 127]
        m = jnp.where(row <= bb, m, 0.0)
        y = lax.dot_general(m, w, (((1,), (1,)), ((), ())),
                            preferred_element_type=jnp.float32) + bias
        o_ref[t * _MAXS:t * _MAXS + _NSEG, :] = y
        o_ref[t * _MAXS + _NSEG:(t + 1) * _MAXS, :] = pad


_tc_linear = pl.pallas_call(
    _tc_body,
    grid=(_B // 2,),
    in_specs=[
        pl.BlockSpec((_NT, 2, _NROW, _D), lambda i: (0, i, 0, 0)),
        pl.BlockSpec((_D, _D), lambda i: (0, 0)),
        pl.BlockSpec((1, _D), lambda i: (0, 0)),
        pl.BlockSpec((_B, 128), lambda i: (0, _L // 128 - 1)),
    ],
    out_specs=pl.BlockSpec((2 * _MAXS, _D), lambda i: (i, 0)),
    out_shape=jax.ShapeDtypeStruct((_B * _MAXS, _D), jnp.float32),
)


def kernel(word_feature, sentence_mask, device, W, b):
    ids = sentence_mask.reshape(_B, _L).astype(jnp.int32)
    part = _sc_segmax(word_feature, ids)
    out = _tc_linear(part, W, b.reshape(1, _D), ids)
    return out.reshape(_B, _MAXS, _D)


# speedup vs baseline: 2.1178x; 1.0398x over previous
import functools

import jax
import jax.numpy as jnp
from jax import lax
from jax.experimental import pallas as pl
from jax.experimental.pallas import tpu as pltpu
from jax.experimental.pallas import tpu_sc as plsc

_B, _L, _D, _MAXS, _NSEG = 4, 4096, 768, 100, 50
_NT = 8
_TPT = _L // _NT
_TCH = 32
_NCH = _TPT // _TCH
_NTRI = (_NCH - 1) // 3
_LN = 16
_GPC = _TCH // _LN
_KV = _D // _LN
_NROW = _NSEG + 8

_mesh = plsc.VectorSubcoreMesh(core_axis_name="c", subcore_axis_name="s")


@functools.partial(
    pl.kernel,
    out_type=jax.ShapeDtypeStruct((_NT, _B, _NROW, _D), jnp.float32),
    mesh=_mesh,
    scratch_types=[
        pltpu.VMEM((_TCH, _D), jnp.float32),
        pltpu.VMEM((_TCH, _D), jnp.float32),
        pltpu.VMEM((_TCH, _D), jnp.float32),
        pltpu.VMEM((_TPT,), jnp.int32),
        pltpu.VMEM((_NROW, _D), jnp.float32),
        pltpu.SemaphoreType.DMA,
        pltpu.SemaphoreType.DMA,
        pltpu.SemaphoreType.DMA,
    ],
    compiler_params=pltpu.CompilerParams(needs_layout_passes=False),
)
def _sc_segmax(wf_hbm, ids_hbm, part_hbm, x0_v, x1_v, x2_v, ids_v, acc_v,
               sem0, sem1, sem2):
    cid = lax.axis_index("c")
    sid = lax.axis_index("s")
    wid = sid * 2 + cid
    b = wid // _NT
    e = wid % _NT
    t0 = e * _TPT

    neg_inf = jnp.full((_LN,), -jnp.inf, jnp.float32)
    bufs = (x0_v, x1_v, x2_v)
    sems = (sem0, sem1, sem2)

    def _src(c):
        return wf_hbm.at[b, pl.ds(t0 + c * _TCH, _TCH), :]

    def _wait(c, u):
        pltpu.make_async_copy(_src(c), bufs[u], sems[u]).wait()

    def _issue(c, u):
        pltpu.async_copy(_src(c), bufs[u], sems[u])

    _issue(0, 0)
    _issue(1, 1)
    _issue(2, 2)

    pltpu.sync_copy(ids_hbm.at[b, pl.ds(t0, _TPT)], ids_v)

    def _init_row(i, carry):
        for k in range(_KV):
            acc_v[i, pl.ds(k * _LN, _LN)] = neg_inf
        return carry

    lax.fori_loop(0, _NROW, _init_row, 0)

    def _compute(c, x_v):
        idg = [ids_v[pl.ds(c * _TCH + g * _LN, _LN)] for g in range(_GPC)]
        mn = jnp.min(idg[0])
        mx = jnp.max(idg[-1])

        def _seg(s, carry):
            sp = jnp.full((_LN,), s, jnp.int32)
            st = jnp.sum((idg[0] < sp).astype(jnp.int32))
            en = jnp.sum((idg[0] <= sp).astype(jnp.int32))
            for g in range(1, _GPC):
                st = st + jnp.sum((idg[g] < sp).astype(jnp.int32))
                en = en + jnp.sum((idg[g] <= sp).astype(jnp.int32))

            def _tok(t, accs):
                return tuple(
                    jnp.maximum(a, x_v[t, pl.ds(k * _LN, _LN)])
                    for k, a in enumerate(accs))

            accs = lax.fori_loop(st, en, _tok, (neg_inf,) * _KV)
            r = jnp.where(s == 0, _NSEG, s - 1)
            for k in range(_KV):
                col = pl.ds(k * _LN, _LN)
                acc_v[r, col] = jnp.maximum(acc_v[r, col], accs[k])
            return carry

        lax.fori_loop(mn, mx + 1, _seg, 0)

    def _triple(ct, carry):
        c0 = 3 * ct
        for u in range(3):
            _wait(c0 + u, u)
            _compute(c0 + u, bufs[u])
            if u == 0:
                _issue(c0 + 3, 0)
            else:
                @pl.when(ct < _NTRI - 1)
                def _(u=u):
                    _issue(c0 + u + 3, u)

        return carry

    lax.fori_loop(0, _NTRI, _triple, 0)
    _wait(_NCH - 1, (_NCH - 1) % 3)
    _compute(_NCH - 1, bufs[(_NCH - 1) % 3])

    pltpu.sync_copy(acc_v, part_hbm.at[e, b])


def _tc_body(p_ref, w_ref, b_ref, ids_ref, o_ref):
    w = w_ref[...]
    bias = b_ref[...]
    pad = jnp.broadcast_to(bias, (_MAXS - _NSEG, _D))
    j = pl.program_id(0)
    row = lax.broadcasted_iota(jnp.int32, (_NSEG, 1), 0) + 1
    for t in range(2):
        m = p_ref[0, t, :_NSEG, :]
        for e in range(1, _NT):
            m = jnp.maximum(m, p_ref[e, t, :_NSEG, :])
        bb = ids_ref[2 * j + t, 127]
        m = jnp.where(row <= bb, m, 0.0)
        y = lax.dot_general(m, w, (((1,), (1,)), ((), ())),
                            preferred_element_type=jnp.float32) + bias
        o_ref[t * _MAXS:t * _MAXS + _NSEG, :] = y
        o_ref[t * _MAXS + _NSEG:(t + 1) * _MAXS, :] = pad


_tc_linear = pl.pallas_call(
    _tc_body,
    grid=(_B // 2,),
    in_specs=[
        pl.BlockSpec((_NT, 2, _NROW, _D), lambda i: (0, i, 0, 0)),
        pl.BlockSpec((_D, _D), lambda i: (0, 0)),
        pl.BlockSpec((1, _D), lambda i: (0, 0)),
        pl.BlockSpec((_B, 128), lambda i: (0, _L // 128 - 1)),
    ],
    out_specs=pl.BlockSpec((2 * _MAXS, _D), lambda i: (i, 0)),
    out_shape=jax.ShapeDtypeStruct((_B * _MAXS, _D), jnp.float32),
)


def kernel(word_feature, sentence_mask, device, W, b):
    ids = sentence_mask.reshape(_B, _L).astype(jnp.int32)
    part = _sc_segmax(word_feature, ids)
    out = _tc_linear(part, W, b.reshape(1, _D), ids)
    return out.reshape(_B, _MAXS, _D)
